# Initial kernel scaffold; baseline (speedup 1.0000x reference)
import jax
import jax.numpy as jnp
from jax.experimental import pallas as pl


def kernel(x, edge_index_0, edge_index_1, W0, W1, Wl, bl):
    raise NotImplementedError("write your pallas kernel here")


# speedup vs baseline: 12.0959x; 12.0959x over previous
import functools
import jax
import jax.numpy as jnp
from jax import lax
from jax.experimental import pallas as pl
from jax.experimental.pallas import tpu as pltpu
from jax.experimental.pallas import tpu_sc as plsc

N = 10000
E = 320000
F = 128
H = 64
G = 8
O = 32

NC = 2
NS = 16
NW = NC * NS
L = 16

NPAD = 10240
RPT = NPAD // NS
CB = 80
EPT = E // NW
EPS = E // NS

_mesh = lambda: plsc.VectorSubcoreMesh(core_axis_name="c", subcore_axis_name="s")


def _counts_call(dst0, dst1):
    @functools.partial(
        pl.kernel,
        out_type=[
            jax.ShapeDtypeStruct((NPAD, 16), jnp.float32),
            jax.ShapeDtypeStruct((NPAD, 16), jnp.float32),
        ],
        mesh=_mesh(),
        compiler_params=pltpu.CompilerParams(
            use_tc_tiling_on_sc=False, needs_layout_passes=False),
        scratch_types=[
            pltpu.VMEM_SHARED((NPAD, 16), jnp.float32),
            pltpu.VMEM((CB,), jnp.int32),
            pltpu.VMEM((CB, 16), jnp.float32),
            pltpu.VMEM((64, 16), jnp.float32),
        ],
    )
    def k(d0_hbm, d1_hbm, c0_hbm, c1_hbm, cnt_sh, dst_buf, oh_buf, zbuf):
        cid = lax.axis_index("c")
        sid = lax.axis_index("s")
        zrow = jnp.zeros((L,), jnp.float32)
        for r in range(64):
            zbuf[r, :] = zrow
        for t in range(RPT // 64):
            pltpu.sync_copy(zbuf, cnt_sh.at[pl.ds(sid * RPT + t * 64, 64)])
        lanes = lax.iota(jnp.int32, 16)
        for e in range(CB):
            oh_buf[e, :] = jnp.where(lanes == (e % G), 1.0, 0.0).astype(jnp.float32)
        plsc.subcore_barrier()

        def run(dref, oref):
            def chunk(c, carry):
                base = sid * EPS + c * CB
                pltpu.sync_copy(dref.at[pl.ds(base, CB)], dst_buf)
                pltpu.sync_copy(oh_buf, cnt_sh.at[dst_buf], add=True)
                return carry

            lax.fori_loop(0, EPS // CB, chunk, 0)
            plsc.subcore_barrier()
            pltpu.sync_copy(cnt_sh.at[pl.ds(sid * RPT, RPT)],
                            oref.at[pl.ds(sid * RPT, RPT)])

        @pl.when(cid == 0)
        def _():
            run(d0_hbm, c0_hbm)

        @pl.when(cid == 1)
        def _():
            run(d1_hbm, c1_hbm)

    return k(dst0, dst1)


def _recip_call(cnt0, cnt1):
    def body(c0, c1, i0, i1):
        i0[...] = 1.0 / jnp.maximum(c0[...], 1.0)
        s = jnp.sum(c1[...], axis=1, keepdims=True)
        i1[...] = jnp.broadcast_to(1.0 / jnp.maximum(s, 1.0), c1.shape)

    blk = NPAD // 8
    return pl.pallas_call(
        body,
        grid=(8,),
        in_specs=[
            pl.BlockSpec((blk, 16), lambda i: (i, 0)),
            pl.BlockSpec((blk, 16), lambda i: (i, 0)),
        ],
        out_specs=[
            pl.BlockSpec((blk, 16), lambda i: (i, 0)),
            pl.BlockSpec((blk, 16), lambda i: (i, 0)),
        ],
        out_shape=[
            jax.ShapeDtypeStruct((NPAD, 16), jnp.float32),
            jax.ShapeDtypeStruct((NPAD, 16), jnp.float32),
        ],
    )(cnt0, cnt1)


def _proj_call(x, w0r):
    def body(x_ref, w_ref, y_ref):
        y_ref[0] = jnp.dot(x_ref[...], w_ref[0], preferred_element_type=jnp.float32)

    return pl.pallas_call(
        body,
        grid=(G, 10),
        in_specs=[
            pl.BlockSpec((1000, F), lambda g, i: (i, 0)),
            pl.BlockSpec((1, F, H), lambda g, i: (g, 0, 0)),
        ],
        out_specs=pl.BlockSpec((1, 1000, H), lambda g, i: (g, i, 0)),
        out_shape=jax.ShapeDtypeStruct((G, N, H), jnp.float32),
    )(x, w0r)


def _agg0_call(y, src0, dst0, inv0):
    @functools.partial(
        pl.kernel,
        out_type=jax.ShapeDtypeStruct((NC, NPAD, H), jnp.float32),
        mesh=_mesh(),
        compiler_params=pltpu.CompilerParams(
            use_tc_tiling_on_sc=False, needs_layout_passes=False),
        scratch_types=[
            pltpu.VMEM_SHARED((NPAD, H), jnp.float32),
            pltpu.VMEM((G * N,), jnp.float32),
            pltpu.VMEM((CB,), jnp.int32),
            pltpu.VMEM((CB,), jnp.int32),
            pltpu.VMEM((CB,), jnp.int32),
            pltpu.VMEM((CB,), jnp.float32),
            pltpu.VMEM((CB, H), jnp.float32),
            pltpu.VMEM((64, H), jnp.float32),
        ],
    )
    def k(y_hbm, s_hbm, d_hbm, inv_hbm, out_hbm,
          acc_sh, inv_buf, src_buf, dst_buf, ridx_buf, sc_buf, rows_buf, zbuf):
        cid = lax.axis_index("c")
        sid = lax.axis_index("s")
        wid = sid * NC + cid
        zrow = jnp.zeros((L,), jnp.float32)
        for r in range(64):
            for j in range(H // L):
                zbuf[r, pl.ds(j * L, L)] = zrow
        for t in range(RPT // 64):
            pltpu.sync_copy(zbuf, acc_sh.at[pl.ds(sid * RPT + t * 64, 64)])
        pltpu.sync_copy(inv_hbm, inv_buf)
        plsc.subcore_barrier()

        gv = lax.iota(jnp.int32, 16) & (G - 1)

        def chunk(c, carry):
            base = wid * EPT + c * CB
            pltpu.sync_copy(s_hbm.at[pl.ds(base, CB)], src_buf)
            pltpu.sync_copy(d_hbm.at[pl.ds(base, CB)], dst_buf)
            for kk in range(CB // L):
                sv = src_buf[pl.ds(kk * L, L)]
                dv = dst_buf[pl.ds(kk * L, L)]
                ridx_buf[pl.ds(kk * L, L)] = gv * N + sv
                sc_buf[pl.ds(kk * L, L)] = plsc.load_gather(inv_buf, [dv * G + gv])
            pltpu.sync_copy(y_hbm.at[ridx_buf], rows_buf)
            for e in range(CB):
                s = plsc.load_gather(sc_buf, [jnp.full((L,), e, jnp.int32)])
                for j in range(H // L):
                    rows_buf[e, pl.ds(j * L, L)] = rows_buf[e, pl.ds(j * L, L)] * s
            pltpu.sync_copy(rows_buf, acc_sh.at[dst_buf], add=True)
            return carry

        lax.fori_loop(0, EPT // CB, chunk, 0)
        plsc.subcore_barrier()
        pltpu.sync_copy(acc_sh.at[pl.ds(sid * RPT, RPT)],
                        out_hbm.at[cid, pl.ds(sid * RPT, RPT)])

    return k(y, src0, dst0, inv0)


def _mid_call(hpart, w1, wl):
    def body(hp_ref, w1_ref, wl_ref, z_ref):
        h = jnp.maximum(hp_ref[0] + hp_ref[1], 0.0)
        z_ref[...] = jnp.dot(
            jnp.dot(h, w1_ref[...], preferred_element_type=jnp.float32),
            wl_ref[...], preferred_element_type=jnp.float32)

    return pl.pallas_call(
        body,
        grid=(10,),
        in_specs=[
            pl.BlockSpec((NC, 1000, H), lambda i: (0, i, 0)),
            pl.BlockSpec((H, H), lambda i: (0, 0)),
            pl.BlockSpec((H, O), lambda i: (0, 0)),
        ],
        out_specs=pl.BlockSpec((1000, O), lambda i: (i, 0)),
        out_shape=jax.ShapeDtypeStruct((N, O), jnp.float32),
    )(hpart, w1, wl)


def _agg1_call(z, src1, dst1, inv1):
    @functools.partial(
        pl.kernel,
        out_type=jax.ShapeDtypeStruct((NC, NPAD, O), jnp.float32),
        mesh=_mesh(),
        compiler_params=pltpu.CompilerParams(
            use_tc_tiling_on_sc=False, needs_layout_passes=False),
        scratch_types=[
            pltpu.VMEM_SHARED((NPAD, O), jnp.float32),
            pltpu.VMEM((N,), jnp.float32),
            pltpu.VMEM((CB,), jnp.int32),
            pltpu.VMEM((CB,), jnp.int32),
            pltpu.VMEM((CB,), jnp.float32),
            pltpu.VMEM((CB, O), jnp.float32),
            pltpu.VMEM((64, O), jnp.float32),
        ],
    )
    def k(z_hbm, s_hbm, d_hbm, inv_hbm, out_hbm,
          acc_sh, inv_buf, src_buf, dst_buf, sc_buf, rows_buf, zbuf):
        cid = lax.axis_index("c")
        sid = lax.axis_index("s")
        wid = sid * NC + cid
        zrow = jnp.zeros((L,), jnp.float32)
        for r in range(64):
            for j in range(O // L):
                zbuf[r, pl.ds(j * L, L)] = zrow
        for t in range(RPT // 64):
            pltpu.sync_copy(zbuf, acc_sh.at[pl.ds(sid * RPT + t * 64, 64)])
        pltpu.sync_copy(inv_hbm, inv_buf)
        plsc.subcore_barrier()

        def chunk(c, carry):
            base = wid * EPT + c * CB
            pltpu.sync_copy(s_hbm.at[pl.ds(base, CB)], src_buf)
            pltpu.sync_copy(d_hbm.at[pl.ds(base, CB)], dst_buf)
            for kk in range(CB // L):
                dv = dst_buf[pl.ds(kk * L, L)]
                sc_buf[pl.ds(kk * L, L)] = plsc.load_gather(inv_buf, [dv])
            pltpu.sync_copy(z_hbm.at[src_buf], rows_buf)
            for e in range(CB):
                s = plsc.load_gather(sc_buf, [jnp.full((L,), e, jnp.int32)])
                for j in range(O // L):
                    rows_buf[e, pl.ds(j * L, L)] = rows_buf[e, pl.ds(j * L, L)] * s
            pltpu.sync_copy(rows_buf, acc_sh.at[dst_buf], add=True)
            return carry

        lax.fori_loop(0, EPT // CB, chunk, 0)
        plsc.subcore_barrier()
        pltpu.sync_copy(acc_sh.at[pl.ds(sid * RPT, RPT)],
                        out_hbm.at[cid, pl.ds(sid * RPT, RPT)])

    return k(z, src1, dst1, inv1)


def _head_call(zpart, bl2):
    def body(zp_ref, b_ref, o_ref):
        s = zp_ref[0] + zp_ref[1] + b_ref[...]
        m = jnp.max(s, axis=1, keepdims=True)
        e = jnp.exp(s - m)
        o_ref[...] = (s - m) - jnp.log(jnp.sum(e, axis=1, keepdims=True))

    return pl.pallas_call(
        body,
        grid=(10,),
        in_specs=[
            pl.BlockSpec((NC, 1000, O), lambda i: (0, i, 0)),
            pl.BlockSpec((1, O), lambda i: (0, 0)),
        ],
        out_specs=pl.BlockSpec((1000, O), lambda i: (i, 0)),
        out_shape=jax.ShapeDtypeStruct((N, O), jnp.float32),
    )(zpart, bl2)


def kernel(x, edge_index_0, edge_index_1, W0, W1, Wl, bl):
    src0, dst0 = edge_index_0[0], edge_index_0[1]
    src1, dst1 = edge_index_1[0], edge_index_1[1]

    cnt0, cnt1 = _counts_call(dst0, dst1)
    inv0f, inv1f = _recip_call(cnt0, cnt1)
    inv0 = inv0f[:N, :G].reshape(N * G)
    inv1 = inv1f[:N, 0]

    y = _proj_call(x, W0.reshape(G, F, H)).reshape(G * N, H)
    hpart = _agg0_call(y, src0, dst0, inv0)
    z = _mid_call(hpart[:, :N, :], W1, Wl)
    zpart = _agg1_call(z, src1, dst1, inv1)
    return _head_call(zpart[:, :N, :], bl.reshape(1, O))


# speedup vs baseline: 20.6733x; 1.7091x over previous
import functools
import jax
import jax.numpy as jnp
from jax import lax
from jax.experimental import pallas as pl
from jax.experimental.pallas import tpu as pltpu
from jax.experimental.pallas import tpu_sc as plsc

N = 10000
E = 320000
F = 128
H = 64
G = 8
O = 32

NC = 2
NS = 16
NW = NC * NS
L = 16

NPAD = 10240
RPT = NPAD // NS
CB = 80
EPT = E // NW
EPS = E // NS
NCH = EPT // CB
NCHC = EPS // CB

_mesh = lambda: plsc.VectorSubcoreMesh(core_axis_name="c", subcore_axis_name="s")
_sc_params = lambda: pltpu.CompilerParams(
    use_tc_tiling_on_sc=False, needs_layout_passes=False)


def _zero_rows(zbuf, width, dst_sh, row0):
    zrow = jnp.zeros((L,), jnp.float32)
    for r in range(64):
        for j in range(width // L):
            zbuf[r, pl.ds(j * L, L)] = zrow
    for t in range(RPT // 64):
        pltpu.sync_copy(zbuf, dst_sh.at[pl.ds(row0 + t * 64, 64)])


def _counts_call(d0_2d, d1_2d):
    @functools.partial(
        pl.kernel,
        out_type=[
            jax.ShapeDtypeStruct((NPAD, 16), jnp.float32),
            jax.ShapeDtypeStruct((NPAD, 16), jnp.float32),
        ],
        mesh=_mesh(),
        compiler_params=_sc_params(),
        scratch_types=[
            pltpu.VMEM_SHARED((NPAD, 16), jnp.float32),
            pltpu.VMEM((NCHC, CB), jnp.int32),
            pltpu.VMEM((CB, 16), jnp.float32),
            pltpu.VMEM((64, 16), jnp.float32),
            pltpu.SemaphoreType.DMA,
        ],
    )
    def k(d0_hbm, d1_hbm, c0_hbm, c1_hbm, cnt_sh, dst2d, oh_buf, zbuf, sem):
        cid = lax.axis_index("c")
        sid = lax.axis_index("s")
        _zero_rows(zbuf, 16, cnt_sh, sid * RPT)
        lanes = lax.iota(jnp.int32, 16)
        for e in range(CB):
            oh_buf[e, :] = jnp.where(lanes == (e % G), 1.0, 0.0).astype(jnp.float32)
        plsc.subcore_barrier()

        def run(dref, oref):
            pltpu.sync_copy(dref.at[pl.ds(sid * NCHC, NCHC)], dst2d)

            def batch(i, carry):
                descs = [
                    pltpu.async_copy(
                        oh_buf, cnt_sh.at[dst2d.at[i * 10 + b]], sem, add=True)
                    for b in range(10)
                ]
                for d in descs:
                    d.wait()
                return carry

            lax.fori_loop(0, NCHC // 10, batch, 0)
            plsc.subcore_barrier()
            pltpu.sync_copy(cnt_sh.at[pl.ds(sid * RPT, RPT)],
                            oref.at[pl.ds(sid * RPT, RPT)])

        @pl.when(cid == 0)
        def _():
            run(d0_hbm, c0_hbm)

        @pl.when(cid == 1)
        def _():
            run(d1_hbm, c1_hbm)

    return k(d0_2d, d1_2d)


def _recip_call(cnt0, cnt1):
    def body(c0, c1, i0, i1):
        i0[...] = 1.0 / jnp.maximum(c0[...], 1.0)
        s = jnp.sum(c1[...], axis=1, keepdims=True)
        i1[...] = jnp.broadcast_to(1.0 / jnp.maximum(s, 1.0), c1.shape)

    blk = NPAD // 8
    return pl.pallas_call(
        body,
        grid=(8,),
        in_specs=[
            pl.BlockSpec((blk, 16), lambda i: (i, 0)),
            pl.BlockSpec((blk, 16), lambda i: (i, 0)),
        ],
        out_specs=[
            pl.BlockSpec((blk, 16), lambda i: (i, 0)),
            pl.BlockSpec((blk, 16), lambda i: (i, 0)),
        ],
        out_shape=[
            jax.ShapeDtypeStruct((NPAD, 16), jnp.float32),
            jax.ShapeDtypeStruct((NPAD, 16), jnp.float32),
        ],
    )(cnt0, cnt1)


def _proj_call(x, w0r):
    def body(x_ref, w_ref, y_ref):
        y_ref[0] = jnp.dot(x_ref[...], w_ref[0], preferred_element_type=jnp.float32)

    return pl.pallas_call(
        body,
        grid=(G, 10),
        in_specs=[
            pl.BlockSpec((1000, F), lambda g, i: (i, 0)),
            pl.BlockSpec((1, F, H), lambda g, i: (g, 0, 0)),
        ],
        out_specs=pl.BlockSpec((1, 1000, H), lambda g, i: (g, i, 0)),
        out_shape=jax.ShapeDtypeStruct((G, N, H), jnp.float32),
    )(x, w0r)


def _agg0_call(y, src0, d0_2d, inv2d):
    @functools.partial(
        pl.kernel,
        out_type=jax.ShapeDtypeStruct((NC, NPAD, H), jnp.float32),
        mesh=_mesh(),
        compiler_params=_sc_params(),
        scratch_types=[
            pltpu.VMEM_SHARED((NPAD, H), jnp.float32),
            pltpu.VMEM((EPT,), jnp.int32),
            pltpu.VMEM((NCH, CB), jnp.int32),
            pltpu.VMEM((CB, H), jnp.float32),
            pltpu.VMEM((CB, H), jnp.float32),
            pltpu.VMEM((CB, 16), jnp.float32),
            pltpu.VMEM((CB, 16), jnp.float32),
            pltpu.VMEM((64, H), jnp.float32),
            pltpu.SemaphoreType.DMA,
            pltpu.SemaphoreType.DMA,
            pltpu.SemaphoreType.DMA,
            pltpu.SemaphoreType.DMA,
        ],
    )
    def k(y_hbm, s_hbm, d2d_hbm, inv_hbm, out_hbm,
          acc_sh, ridx, dst2d, rows0, rows1, invr0, invr1, zbuf,
          gsem0, gsem1, ssem0, ssem1):
        cid = lax.axis_index("c")
        sid = lax.axis_index("s")
        wid = sid * NC + cid
        _zero_rows(zbuf, H, acc_sh, sid * RPT)
        pltpu.sync_copy(d2d_hbm.at[pl.ds(wid * NCH, NCH)], dst2d)

        gv = lax.iota(jnp.int32, 16) & (G - 1)

        pltpu.sync_copy(s_hbm.at[pl.ds(wid * EPT, EPT)], ridx)

        def mk_ridx(i, carry):
            ridx[pl.ds(i * L, L)] = gv * N + ridx[pl.ds(i * L, L)]
            return carry

        lax.fori_loop(0, EPT // L, mk_ridx, 0)
        plsc.subcore_barrier()

        def start_gather(c, rbuf, ibuf, sem):
            pltpu.async_copy(y_hbm.at[ridx.at[pl.ds(c * CB, CB)]], rbuf, sem)
            pltpu.async_copy(inv_hbm.at[dst2d.at[c]], ibuf, sem)

        def wait_gather(c, rbuf, ibuf, sem):
            pltpu.make_async_copy(
                y_hbm.at[ridx.at[pl.ds(c * CB, CB)]], rbuf, sem).wait()
            pltpu.make_async_copy(inv_hbm.at[dst2d.at[c]], ibuf, sem).wait()

        def scale(rbuf, ibuf):
            for e in range(CB):
                s = plsc.load_gather(
                    ibuf,
                    [jnp.full((L,), e, jnp.int32), jnp.full((L,), e % G, jnp.int32)])
                for j in range(H // L):
                    rbuf[e, pl.ds(j * L, L)] = rbuf[e, pl.ds(j * L, L)] * s

        start_gather(0, rows0, invr0, gsem0)
        start_gather(1, rows1, invr1, gsem1)

        def pair(i, carry):
            c0 = 2 * i
            wait_gather(c0, rows0, invr0, gsem0)
            scale(rows0, invr0)
            s0 = pltpu.async_copy(rows0, acc_sh.at[dst2d.at[c0]], ssem0, add=True)
            wait_gather(c0 + 1, rows1, invr1, gsem1)
            scale(rows1, invr1)
            s1 = pltpu.async_copy(rows1, acc_sh.at[dst2d.at[c0 + 1]], ssem1, add=True)
            s0.wait()
            start_gather(c0 + 2, rows0, invr0, gsem0)

            @pl.when(i < (NCH - 1) // 2 - 1)
            def _():
                start_gather(c0 + 3, rows1, invr1, gsem1)

            s1.wait()
            return carry

        lax.fori_loop(0, (NCH - 1) // 2, pair, 0)
        cl = NCH - 1
        wait_gather(cl, rows0, invr0, gsem0)
        scale(rows0, invr0)
        pltpu.sync_copy(rows0, acc_sh.at[dst2d.at[cl]], add=True)

        plsc.subcore_barrier()
        pltpu.sync_copy(acc_sh.at[pl.ds(sid * RPT, RPT)],
                        out_hbm.at[cid, pl.ds(sid * RPT, RPT)])

    return k(y, src0, d0_2d, inv2d)


def _mid_call(hpart, w1, wl):
    def body(hp_ref, w1_ref, wl_ref, z_ref):
        h = jnp.maximum(hp_ref[0] + hp_ref[1], 0.0)
        z_ref[...] = jnp.dot(
            jnp.dot(h, w1_ref[...], preferred_element_type=jnp.float32),
            wl_ref[...], preferred_element_type=jnp.float32)

    return pl.pallas_call(
        body,
        grid=(10,),
        in_specs=[
            pl.BlockSpec((NC, 1000, H), lambda i: (0, i, 0)),
            pl.BlockSpec((H, H), lambda i: (0, 0)),
            pl.BlockSpec((H, O), lambda i: (0, 0)),
        ],
        out_specs=pl.BlockSpec((1000, O), lambda i: (i, 0)),
        out_shape=jax.ShapeDtypeStruct((N, O), jnp.float32),
    )(hpart, w1, wl)


def _agg1_call(z, src1, d1_2d):
    @functools.partial(
        pl.kernel,
        out_type=jax.ShapeDtypeStruct((NC, NPAD, O), jnp.float32),
        mesh=_mesh(),
        compiler_params=_sc_params(),
        scratch_types=[
            pltpu.VMEM_SHARED((NPAD, O), jnp.float32),
            pltpu.VMEM((EPT,), jnp.int32),
            pltpu.VMEM((NCH, CB), jnp.int32),
            pltpu.VMEM((CB, O), jnp.float32),
            pltpu.VMEM((CB, O), jnp.float32),
            pltpu.VMEM((64, O), jnp.float32),
            pltpu.SemaphoreType.DMA,
            pltpu.SemaphoreType.DMA,
            pltpu.SemaphoreType.DMA,
            pltpu.SemaphoreType.DMA,
        ],
    )
    def k(z_hbm, s_hbm, d2d_hbm, out_hbm,
          acc_sh, sidx, dst2d, rows0, rows1, zbuf,
          gsem0, gsem1, ssem0, ssem1):
        cid = lax.axis_index("c")
        sid = lax.axis_index("s")
        wid = sid * NC + cid
        _zero_rows(zbuf, O, acc_sh, sid * RPT)
        pltpu.sync_copy(s_hbm.at[pl.ds(wid * EPT, EPT)], sidx)
        pltpu.sync_copy(d2d_hbm.at[pl.ds(wid * NCH, NCH)], dst2d)
        plsc.subcore_barrier()

        def start_gather(c, rbuf, sem):
            return pltpu.async_copy(
                z_hbm.at[sidx.at[pl.ds(c * CB, CB)]], rbuf, sem)

        start_gather(0, rows0, gsem0)
        start_gather(1, rows1, gsem1)

        def pair(i, carry):
            c0 = 2 * i
            pltpu.make_async_copy(
                z_hbm.at[sidx.at[pl.ds(c0 * CB, CB)]], rows0, gsem0).wait()
            s0 = pltpu.async_copy(rows0, acc_sh.at[dst2d.at[c0]], ssem0, add=True)
            pltpu.make_async_copy(
                z_hbm.at[sidx.at[pl.ds((c0 + 1) * CB, CB)]], rows1, gsem1).wait()
            s1 = pltpu.async_copy(rows1, acc_sh.at[dst2d.at[c0 + 1]], ssem1, add=True)
            s0.wait()
            start_gather(c0 + 2, rows0, gsem0)

            @pl.when(i < (NCH - 1) // 2 - 1)
            def _():
                start_gather(c0 + 3, rows1, gsem1)

            s1.wait()
            return carry

        lax.fori_loop(0, (NCH - 1) // 2, pair, 0)
        cl = NCH - 1
        pltpu.make_async_copy(
            z_hbm.at[sidx.at[pl.ds(cl * CB, CB)]], rows0, gsem0).wait()
        pltpu.sync_copy(rows0, acc_sh.at[dst2d.at[cl]], add=True)

        plsc.subcore_barrier()
        pltpu.sync_copy(acc_sh.at[pl.ds(sid * RPT, RPT)],
                        out_hbm.at[cid, pl.ds(sid * RPT, RPT)])

    return k(z, src1, d1_2d)


def _head_call(zpart, inv1f, bl2):
    def body(zp_ref, inv_ref, b_ref, o_ref):
        s = (zp_ref[0] + zp_ref[1]) * inv_ref[...][:, :1] + b_ref[...]
        m = jnp.max(s, axis=1, keepdims=True)
        e = jnp.exp(s - m)
        o_ref[...] = (s - m) - jnp.log(jnp.sum(e, axis=1, keepdims=True))

    return pl.pallas_call(
        body,
        grid=(10,),
        in_specs=[
            pl.BlockSpec((NC, 1000, O), lambda i: (0, i, 0)),
            pl.BlockSpec((1000, 16), lambda i: (i, 0)),
            pl.BlockSpec((1, O), lambda i: (0, 0)),
        ],
        out_specs=pl.BlockSpec((1000, O), lambda i: (i, 0)),
        out_shape=jax.ShapeDtypeStruct((N, O), jnp.float32),
    )(zpart, inv1f, bl2)


def kernel(x, edge_index_0, edge_index_1, W0, W1, Wl, bl):
    src0, dst0 = edge_index_0[0], edge_index_0[1]
    src1, dst1 = edge_index_1[0], edge_index_1[1]
    d0_2d = dst0.reshape(E // CB, CB)
    d1_2d = dst1.reshape(E // CB, CB)

    cnt0, cnt1 = _recip_call(*_counts_call(d0_2d, d1_2d))
    inv0f, inv1f = cnt0, cnt1

    y = _proj_call(x, W0.reshape(G, F, H)).reshape(G * N, H)
    hpart = _agg0_call(y, src0, d0_2d, inv0f[:N])
    z = _mid_call(hpart[:, :N, :], W1, Wl)
    zpart = _agg1_call(z, src1, d1_2d)
    return _head_call(zpart[:, :N, :], inv1f[:N], bl.reshape(1, O))


# speedup vs baseline: 28.8721x; 1.3966x over previous
import functools
import jax
import jax.numpy as jnp
from jax import lax
from jax.experimental import pallas as pl
from jax.experimental.pallas import tpu as pltpu
from jax.experimental.pallas import tpu_sc as plsc

N = 10000
E = 320000
F = 128
H = 64
G = 8
O = 32

NC = 2
NS = 16
NW = NC * NS
L = 16

NPAD = 10240
RPT = NPAD // NS
CB = 80
EPT = E // NW
EPS = E // NS
NCH = EPT // CB
NCHC = EPS // CB
D = 5

_mesh = lambda: plsc.VectorSubcoreMesh(core_axis_name="c", subcore_axis_name="s")
_sc_params = lambda: pltpu.CompilerParams(
    use_tc_tiling_on_sc=False, needs_layout_passes=False)


def _zero_rows(zbuf, width, dst_sh, row0):
    zrow = jnp.zeros((L,), jnp.float32)
    for r in range(64):
        for j in range(width // L):
            zbuf[r, pl.ds(j * L, L)] = zrow
    for t in range(RPT // 64):
        pltpu.sync_copy(zbuf, dst_sh.at[pl.ds(row0 + t * 64, 64)])


def _counts_call(d0_2d, d1_2d):
    @functools.partial(
        pl.kernel,
        out_type=[
            jax.ShapeDtypeStruct((NPAD, 16), jnp.float32),
            jax.ShapeDtypeStruct((NPAD, 16), jnp.float32),
        ],
        mesh=_mesh(),
        compiler_params=_sc_params(),
        scratch_types=[
            pltpu.VMEM_SHARED((NPAD, 16), jnp.float32),
            pltpu.VMEM((NCHC, CB), jnp.int32),
            pltpu.VMEM((CB, 16), jnp.float32),
            pltpu.VMEM((64, 16), jnp.float32),
            pltpu.VMEM((RPT, 16), jnp.float32),
            pltpu.SemaphoreType.DMA,
        ],
    )
    def k(d0_hbm, d1_hbm, i0_hbm, i1_hbm, cnt_sh, dst2d, oh_buf, zbuf, ibuf, sem):
        cid = lax.axis_index("c")
        sid = lax.axis_index("s")
        _zero_rows(zbuf, 16, cnt_sh, sid * RPT)
        lanes = lax.iota(jnp.int32, 16)
        for e in range(CB):
            oh_buf[e, :] = jnp.where(lanes == (e % G), 1.0, 0.0).astype(jnp.float32)
        plsc.subcore_barrier()

        def run(dref, iref, sum_cols):
            pltpu.sync_copy(dref.at[pl.ds(sid * NCHC, NCHC)], dst2d)

            def batch(i, carry):
                descs = [
                    pltpu.async_copy(
                        oh_buf, cnt_sh.at[dst2d.at[i * 10 + b]], sem, add=True)
                    for b in range(10)
                ]
                for d in descs:
                    d.wait()
                return carry

            lax.fori_loop(0, NCHC // 10, batch, 0)
            plsc.subcore_barrier()
            pltpu.sync_copy(cnt_sh.at[pl.ds(sid * RPT, RPT)], ibuf)

            if sum_cols:
                def inv_row(r, carry):
                    row = ibuf[r, :]
                    tot = jnp.maximum(jnp.sum(row), 1.0)
                    ibuf[r, :] = jnp.full((L,), 1.0, jnp.float32) / tot
                    return carry
            else:
                def inv_row(r, carry):
                    row = ibuf[r, :]
                    ibuf[r, :] = 1.0 / jnp.maximum(row, 1.0)
                    return carry

            lax.fori_loop(0, RPT, inv_row, 0)
            pltpu.sync_copy(ibuf, iref.at[pl.ds(sid * RPT, RPT)])

        @pl.when(cid == 0)
        def _():
            run(d0_hbm, i0_hbm, False)

        @pl.when(cid == 1)
        def _():
            run(d1_hbm, i1_hbm, True)

    return k(d0_2d, d1_2d)


def _proj_call(x, w0r):
    def body(x_ref, w_ref, y_ref):
        y_ref[0] = jnp.dot(x_ref[...], w_ref[0], preferred_element_type=jnp.float32)

    return pl.pallas_call(
        body,
        grid=(G, 10),
        in_specs=[
            pl.BlockSpec((1000, F), lambda g, i: (i, 0)),
            pl.BlockSpec((1, F, H), lambda g, i: (g, 0, 0)),
        ],
        out_specs=pl.BlockSpec((1, 1000, H), lambda g, i: (g, i, 0)),
        out_shape=jax.ShapeDtypeStruct((G, N, H), jnp.float32),
    )(x, w0r)


def _agg0_call(y, src0, d0_2d, inv2d):
    @functools.partial(
        pl.kernel,
        out_type=jax.ShapeDtypeStruct((NC, NPAD, H), jnp.float32),
        mesh=_mesh(),
        compiler_params=_sc_params(),
        scratch_types=[
            pltpu.VMEM_SHARED((NPAD, H), jnp.float32),
            pltpu.VMEM((EPT,), jnp.int32),
            pltpu.VMEM((NCH, CB), jnp.int32),
            [pltpu.VMEM((CB, H), jnp.float32) for _ in range(D)],
            [pltpu.VMEM((CB, 16), jnp.float32) for _ in range(D)],
            pltpu.VMEM((64, H), jnp.float32),
            [pltpu.SemaphoreType.DMA for _ in range(D)],
            pltpu.SemaphoreType.DMA,
        ],
    )
    def k(y_hbm, s_hbm, d2d_hbm, inv_hbm, out_hbm,
          acc_sh, ridx, dst2d, rows, invr, zbuf, gsems, ssem):
        cid = lax.axis_index("c")
        sid = lax.axis_index("s")
        wid = sid * NC + cid
        _zero_rows(zbuf, H, acc_sh, sid * RPT)
        pltpu.sync_copy(d2d_hbm.at[pl.ds(wid * NCH, NCH)], dst2d)

        gv = lax.iota(jnp.int32, 16) & (G - 1)

        pltpu.sync_copy(s_hbm.at[pl.ds(wid * EPT, EPT)], ridx)

        def mk_ridx(i, carry):
            ridx[pl.ds(i * L, L)] = gv * N + ridx[pl.ds(i * L, L)]
            return carry

        lax.fori_loop(0, EPT // L, mk_ridx, 0)
        plsc.subcore_barrier()

        def start_gather(c, b):
            pltpu.async_copy(y_hbm.at[ridx.at[pl.ds(c * CB, CB)]], rows[b], gsems[b])
            pltpu.async_copy(inv_hbm.at[dst2d.at[c]], invr[b], gsems[b])

        def wait_gather(c, b):
            pltpu.make_async_copy(
                y_hbm.at[ridx.at[pl.ds(c * CB, CB)]], rows[b], gsems[b]).wait()
            pltpu.make_async_copy(inv_hbm.at[dst2d.at[c]], invr[b], gsems[b]).wait()

        def scale(rbuf, ibuf):
            def srow(j, carry):
                for t in range(D):
                    e = j * D + t
                    s = plsc.load_gather(
                        ibuf,
                        [jnp.full((L,), e, jnp.int32),
                         jnp.full((L,), e & (G - 1), jnp.int32)])
                    for jj in range(H // L):
                        rbuf[e, pl.ds(jj * L, L)] = rbuf[e, pl.ds(jj * L, L)] * s
                return carry

            lax.fori_loop(0, CB // D, srow, 0)

        for b in range(D):
            start_gather(b, b)

        def ring(i, carry):
            for b in range(D):
                c = i * D + b
                wait_gather(c, b)
                scale(rows[b], invr[b])
                pltpu.async_copy(rows[b], acc_sh.at[dst2d.at[c]], ssem,
                                 add=True).wait()

                @pl.when(i < NCH // D - 1)
                def _():
                    start_gather(c + D, b)

            return carry

        lax.fori_loop(0, NCH // D, ring, 0)

        plsc.subcore_barrier()
        pltpu.sync_copy(acc_sh.at[pl.ds(sid * RPT, RPT)],
                        out_hbm.at[cid, pl.ds(sid * RPT, RPT)])

    return k(y, src0, d0_2d, inv2d)


def _mid_call(hpart, w1, wl):
    def body(hp_ref, w1_ref, wl_ref, z_ref):
        h = jnp.maximum(hp_ref[0] + hp_ref[1], 0.0)
        z_ref[...] = jnp.dot(
            jnp.dot(h, w1_ref[...], preferred_element_type=jnp.float32),
            wl_ref[...], preferred_element_type=jnp.float32)

    return pl.pallas_call(
        body,
        grid=(10,),
        in_specs=[
            pl.BlockSpec((NC, 1000, H), lambda i: (0, i, 0)),
            pl.BlockSpec((H, H), lambda i: (0, 0)),
            pl.BlockSpec((H, O), lambda i: (0, 0)),
        ],
        out_specs=pl.BlockSpec((1000, O), lambda i: (i, 0)),
        out_shape=jax.ShapeDtypeStruct((N, O), jnp.float32),
    )(hpart, w1, wl)


def _agg1_call(z, src1, d1_2d):
    @functools.partial(
        pl.kernel,
        out_type=jax.ShapeDtypeStruct((NC, NPAD, O), jnp.float32),
        mesh=_mesh(),
        compiler_params=_sc_params(),
        scratch_types=[
            pltpu.VMEM_SHARED((NPAD, O), jnp.float32),
            pltpu.VMEM((EPT,), jnp.int32),
            pltpu.VMEM((NCH, CB), jnp.int32),
            [pltpu.VMEM((CB, O), jnp.float32) for _ in range(D)],
            pltpu.VMEM((64, O), jnp.float32),
            [pltpu.SemaphoreType.DMA for _ in range(D)],
            pltpu.SemaphoreType.DMA,
        ],
    )
    def k(z_hbm, s_hbm, d2d_hbm, out_hbm,
          acc_sh, sidx, dst2d, rows, zbuf, gsems, ssem):
        cid = lax.axis_index("c")
        sid = lax.axis_index("s")
        wid = sid * NC + cid
        _zero_rows(zbuf, O, acc_sh, sid * RPT)
        pltpu.sync_copy(s_hbm.at[pl.ds(wid * EPT, EPT)], sidx)
        pltpu.sync_copy(d2d_hbm.at[pl.ds(wid * NCH, NCH)], dst2d)
        plsc.subcore_barrier()

        def start_gather(c, b):
            pltpu.async_copy(z_hbm.at[sidx.at[pl.ds(c * CB, CB)]], rows[b], gsems[b])

        def wait_gather(c, b):
            pltpu.make_async_copy(
                z_hbm.at[sidx.at[pl.ds(c * CB, CB)]], rows[b], gsems[b]).wait()

        for b in range(D):
            start_gather(b, b)

        def ring(i, carry):
            for b in range(D):
                c = i * D + b
                wait_gather(c, b)
                pltpu.async_copy(rows[b], acc_sh.at[dst2d.at[c]], ssem,
                                 add=True).wait()

                @pl.when(i < NCH // D - 1)
                def _():
                    start_gather(c + D, b)

            return carry

        lax.fori_loop(0, NCH // D, ring, 0)

        plsc.subcore_barrier()
        pltpu.sync_copy(acc_sh.at[pl.ds(sid * RPT, RPT)],
                        out_hbm.at[cid, pl.ds(sid * RPT, RPT)])

    return k(z, src1, d1_2d)


def _head_call(zpart, inv1f, bl2):
    def body(zp_ref, inv_ref, b_ref, o_ref):
        s = (zp_ref[0] + zp_ref[1]) * inv_ref[...][:, :1] + b_ref[...]
        m = jnp.max(s, axis=1, keepdims=True)
        e = jnp.exp(s - m)
        o_ref[...] = (s - m) - jnp.log(jnp.sum(e, axis=1, keepdims=True))

    return pl.pallas_call(
        body,
        grid=(10,),
        in_specs=[
            pl.BlockSpec((NC, 1000, O), lambda i: (0, i, 0)),
            pl.BlockSpec((1000, 16), lambda i: (i, 0)),
            pl.BlockSpec((1, O), lambda i: (0, 0)),
        ],
        out_specs=pl.BlockSpec((1000, O), lambda i: (i, 0)),
        out_shape=jax.ShapeDtypeStruct((N, O), jnp.float32),
    )(zpart, inv1f, bl2)


def kernel(x, edge_index_0, edge_index_1, W0, W1, Wl, bl):
    src0, dst0 = edge_index_0[0], edge_index_0[1]
    src1, dst1 = edge_index_1[0], edge_index_1[1]
    d0_2d = dst0.reshape(E // CB, CB)
    d1_2d = dst1.reshape(E // CB, CB)

    inv0f, inv1f = _counts_call(d0_2d, d1_2d)

    y = _proj_call(x, W0.reshape(G, F, H)).reshape(G * N, H)
    hpart = _agg0_call(y, src0, d0_2d, inv0f[:N])
    z = _mid_call(hpart[:, :N, :], W1, Wl)
    zpart = _agg1_call(z, src1, d1_2d)
    return _head_call(zpart[:, :N, :], inv1f[:N], bl.reshape(1, O))


# speedup vs baseline: 28.9026x; 1.0011x over previous
import functools
import jax
import jax.numpy as jnp
from jax import lax
from jax.experimental import pallas as pl
from jax.experimental.pallas import tpu as pltpu
from jax.experimental.pallas import tpu_sc as plsc

N = 10000
E = 320000
F = 128
H = 64
G = 8
O = 32

NC = 2
NS = 16
NW = NC * NS
L = 16

NPAD = 10240
RPT = NPAD // NS
CB = 80
EPT = E // NW
EPS = E // NS
NCH = EPT // CB
NCHC = EPS // CB
D = 5

_mesh = lambda: plsc.VectorSubcoreMesh(core_axis_name="c", subcore_axis_name="s")
_sc_params = lambda: pltpu.CompilerParams(
    use_tc_tiling_on_sc=False, needs_layout_passes=False)


def _zero_rows(zbuf, width, dst_sh, row0):
    zrow = jnp.zeros((L,), jnp.float32)
    for r in range(64):
        for j in range(width // L):
            zbuf[r, pl.ds(j * L, L)] = zrow
    for t in range(RPT // 64):
        pltpu.sync_copy(zbuf, dst_sh.at[pl.ds(row0 + t * 64, 64)])


def _counts_call(d0_2d, d1_2d):
    @functools.partial(
        pl.kernel,
        out_type=[
            jax.ShapeDtypeStruct((NPAD, 16), jnp.float32),
            jax.ShapeDtypeStruct((NPAD, 16), jnp.float32),
        ],
        mesh=_mesh(),
        compiler_params=_sc_params(),
        scratch_types=[
            pltpu.VMEM_SHARED((NPAD, 16), jnp.float32),
            pltpu.VMEM((NCHC, CB), jnp.int32),
            pltpu.VMEM((CB, 16), jnp.float32),
            pltpu.VMEM((64, 16), jnp.float32),
            pltpu.VMEM((RPT, 16), jnp.float32),
            pltpu.SemaphoreType.DMA,
        ],
    )
    def k(d0_hbm, d1_hbm, i0_hbm, i1_hbm, cnt_sh, dst2d, oh_buf, zbuf, ibuf, sem):
        cid = lax.axis_index("c")
        sid = lax.axis_index("s")
        _zero_rows(zbuf, 16, cnt_sh, sid * RPT)
        lanes = lax.iota(jnp.int32, 16)

        def run(dref, iref, grouped):
            for e in range(CB):
                hot = (e % G) if grouped else 0
                oh_buf[e, :] = jnp.where(lanes == hot, 1.0, 0.0).astype(jnp.float32)
            plsc.subcore_barrier()
            pltpu.sync_copy(dref.at[pl.ds(sid * NCHC, NCHC)], dst2d)

            def batch(i, carry):
                descs = [
                    pltpu.async_copy(
                        oh_buf, cnt_sh.at[dst2d.at[i * 10 + b]], sem, add=True)
                    for b in range(10)
                ]
                for d in descs:
                    d.wait()
                return carry

            lax.fori_loop(0, NCHC // 10, batch, 0)
            plsc.subcore_barrier()
            pltpu.sync_copy(cnt_sh.at[pl.ds(sid * RPT, RPT)], ibuf)

            def inv_row(r, carry):
                ibuf[r, :] = 1.0 / jnp.maximum(ibuf[r, :], 1.0)
                return carry

            lax.fori_loop(0, RPT, inv_row, 0)
            pltpu.sync_copy(ibuf, iref.at[pl.ds(sid * RPT, RPT)])

        @pl.when(cid == 0)
        def _():
            run(d0_hbm, i0_hbm, True)

        @pl.when(cid == 1)
        def _():
            run(d1_hbm, i1_hbm, False)

    return k(d0_2d, d1_2d)


def _proj_call(x, w0r):
    def body(x_ref, w_ref, y_ref):
        y_ref[0] = jnp.dot(x_ref[...], w_ref[0], preferred_element_type=jnp.float32)

    return pl.pallas_call(
        body,
        grid=(G, 10),
        in_specs=[
            pl.BlockSpec((1000, F), lambda g, i: (i, 0)),
            pl.BlockSpec((1, F, H), lambda g, i: (g, 0, 0)),
        ],
        out_specs=pl.BlockSpec((1, 1000, H), lambda g, i: (g, i, 0)),
        out_shape=jax.ShapeDtypeStruct((G, N, H), jnp.float32),
    )(x, w0r)


def _agg0_call(y, src0, d0_2d, inv2d):
    @functools.partial(
        pl.kernel,
        out_type=jax.ShapeDtypeStruct((NC, NPAD, H), jnp.float32),
        mesh=_mesh(),
        compiler_params=_sc_params(),
        scratch_types=[
            pltpu.VMEM_SHARED((NPAD, H), jnp.float32),
            pltpu.VMEM((EPT,), jnp.int32),
            pltpu.VMEM((NCH, CB), jnp.int32),
            [pltpu.VMEM((CB, H), jnp.float32) for _ in range(D)],
            [pltpu.VMEM((CB, 16), jnp.float32) for _ in range(D)],
            pltpu.VMEM((64, H), jnp.float32),
            [pltpu.SemaphoreType.DMA for _ in range(D)],
            pltpu.SemaphoreType.DMA,
        ],
    )
    def k(y_hbm, s_hbm, d2d_hbm, inv_hbm, out_hbm,
          acc_sh, ridx, dst2d, rows, invr, zbuf, gsems, ssem):
        cid = lax.axis_index("c")
        sid = lax.axis_index("s")
        wid = sid * NC + cid
        _zero_rows(zbuf, H, acc_sh, sid * RPT)
        pltpu.sync_copy(d2d_hbm.at[pl.ds(wid * NCH, NCH)], dst2d)

        gv = lax.iota(jnp.int32, 16) & (G - 1)

        pltpu.sync_copy(s_hbm.at[pl.ds(wid * EPT, EPT)], ridx)

        def mk_ridx(i, carry):
            ridx[pl.ds(i * L, L)] = gv * N + ridx[pl.ds(i * L, L)]
            return carry

        lax.fori_loop(0, EPT // L, mk_ridx, 0)
        plsc.subcore_barrier()

        def start_gather(c, b):
            pltpu.async_copy(y_hbm.at[ridx.at[pl.ds(c * CB, CB)]], rows[b], gsems[b])
            pltpu.async_copy(inv_hbm.at[dst2d.at[c]], invr[b], gsems[b])

        def wait_gather(c, b):
            pltpu.make_async_copy(
                y_hbm.at[ridx.at[pl.ds(c * CB, CB)]], rows[b], gsems[b]).wait()
            pltpu.make_async_copy(inv_hbm.at[dst2d.at[c]], invr[b], gsems[b]).wait()

        def scale(rbuf, ibuf):
            def srow(j, carry):
                for t in range(D):
                    e = j * D + t
                    s = plsc.load_gather(
                        ibuf,
                        [jnp.full((L,), e, jnp.int32),
                         jnp.full((L,), e & (G - 1), jnp.int32)])
                    for jj in range(H // L):
                        rbuf[e, pl.ds(jj * L, L)] = rbuf[e, pl.ds(jj * L, L)] * s
                return carry

            lax.fori_loop(0, CB // D, srow, 0)

        for b in range(D):
            start_gather(b, b)

        def ring(i, carry):
            for b in range(D):
                c = i * D + b
                wait_gather(c, b)
                scale(rows[b], invr[b])
                pltpu.async_copy(rows[b], acc_sh.at[dst2d.at[c]], ssem,
                                 add=True).wait()

                @pl.when(i < NCH // D - 1)
                def _():
                    start_gather(c + D, b)

            return carry

        lax.fori_loop(0, NCH // D, ring, 0)

        plsc.subcore_barrier()
        pltpu.sync_copy(acc_sh.at[pl.ds(sid * RPT, RPT)],
                        out_hbm.at[cid, pl.ds(sid * RPT, RPT)])

    return k(y, src0, d0_2d, inv2d)


def _mid_call(hpart, w1, wl):
    def body(hp_ref, w1_ref, wl_ref, z_ref):
        h = jnp.maximum(hp_ref[0] + hp_ref[1], 0.0)
        z_ref[...] = jnp.dot(
            jnp.dot(h, w1_ref[...], preferred_element_type=jnp.float32),
            wl_ref[...], preferred_element_type=jnp.float32)

    return pl.pallas_call(
        body,
        grid=(10,),
        in_specs=[
            pl.BlockSpec((NC, 1000, H), lambda i: (0, i, 0)),
            pl.BlockSpec((H, H), lambda i: (0, 0)),
            pl.BlockSpec((H, O), lambda i: (0, 0)),
        ],
        out_specs=pl.BlockSpec((1000, O), lambda i: (i, 0)),
        out_shape=jax.ShapeDtypeStruct((N, O), jnp.float32),
    )(hpart, w1, wl)


def _agg1_call(z, src1, d1_2d):
    @functools.partial(
        pl.kernel,
        out_type=jax.ShapeDtypeStruct((NC, NPAD, O), jnp.float32),
        mesh=_mesh(),
        compiler_params=_sc_params(),
        scratch_types=[
            pltpu.VMEM_SHARED((NPAD, O), jnp.float32),
            pltpu.VMEM((EPT,), jnp.int32),
            pltpu.VMEM((NCH, CB), jnp.int32),
            [pltpu.VMEM((CB, O), jnp.float32) for _ in range(D)],
            pltpu.VMEM((64, O), jnp.float32),
            [pltpu.SemaphoreType.DMA for _ in range(D)],
            pltpu.SemaphoreType.DMA,
        ],
    )
    def k(z_hbm, s_hbm, d2d_hbm, out_hbm,
          acc_sh, sidx, dst2d, rows, zbuf, gsems, ssem):
        cid = lax.axis_index("c")
        sid = lax.axis_index("s")
        wid = sid * NC + cid
        _zero_rows(zbuf, O, acc_sh, sid * RPT)
        pltpu.sync_copy(s_hbm.at[pl.ds(wid * EPT, EPT)], sidx)
        pltpu.sync_copy(d2d_hbm.at[pl.ds(wid * NCH, NCH)], dst2d)
        plsc.subcore_barrier()

        def start_gather(c, b):
            pltpu.async_copy(z_hbm.at[sidx.at[pl.ds(c * CB, CB)]], rows[b], gsems[b])

        def wait_gather(c, b):
            pltpu.make_async_copy(
                z_hbm.at[sidx.at[pl.ds(c * CB, CB)]], rows[b], gsems[b]).wait()

        for b in range(D):
            start_gather(b, b)

        def ring(i, carry):
            for b in range(D):
                c = i * D + b
                wait_gather(c, b)
                pltpu.async_copy(rows[b], acc_sh.at[dst2d.at[c]], ssem,
                                 add=True).wait()

                @pl.when(i < NCH // D - 1)
                def _():
                    start_gather(c + D, b)

            return carry

        lax.fori_loop(0, NCH // D, ring, 0)

        plsc.subcore_barrier()
        pltpu.sync_copy(acc_sh.at[pl.ds(sid * RPT, RPT)],
                        out_hbm.at[cid, pl.ds(sid * RPT, RPT)])

    return k(z, src1, d1_2d)


def _head_call(zpart, inv1f, bl2):
    def body(zp_ref, inv_ref, b_ref, o_ref):
        s = (zp_ref[0] + zp_ref[1]) * inv_ref[...][:, :1] + b_ref[...]
        m = jnp.max(s, axis=1, keepdims=True)
        e = jnp.exp(s - m)
        o_ref[...] = (s - m) - jnp.log(jnp.sum(e, axis=1, keepdims=True))

    return pl.pallas_call(
        body,
        grid=(10,),
        in_specs=[
            pl.BlockSpec((NC, 1000, O), lambda i: (0, i, 0)),
            pl.BlockSpec((1000, 16), lambda i: (i, 0)),
            pl.BlockSpec((1, O), lambda i: (0, 0)),
        ],
        out_specs=pl.BlockSpec((1000, O), lambda i: (i, 0)),
        out_shape=jax.ShapeDtypeStruct((N, O), jnp.float32),
    )(zpart, inv1f, bl2)


def kernel(x, edge_index_0, edge_index_1, W0, W1, Wl, bl):
    src0, dst0 = edge_index_0[0], edge_index_0[1]
    src1, dst1 = edge_index_1[0], edge_index_1[1]
    d0_2d = dst0.reshape(E // CB, CB)
    d1_2d = dst1.reshape(E // CB, CB)

    inv0f, inv1f = _counts_call(d0_2d, d1_2d)

    y = _proj_call(x, W0.reshape(G, F, H)).reshape(G * N, H)
    hpart = _agg0_call(y, src0, d0_2d, inv0f[:N])
    z = _mid_call(hpart[:, :N, :], W1, Wl)
    zpart = _agg1_call(z, src1, d1_2d)
    return _head_call(zpart[:, :N, :], inv1f[:N], bl.reshape(1, O))
